# Initial kernel scaffold; baseline (speedup 1.0000x reference)
import jax
import jax.numpy as jnp
from jax.experimental import pallas as pl


def kernel(x, A, temperature):
    raise NotImplementedError("write your pallas kernel here")


# speedup vs baseline: 9.2774x; 9.2774x over previous
import functools

import jax
import jax.numpy as jnp
from jax.experimental import pallas as pl
from jax.experimental.pallas import tpu as pltpu

_N = 1000
_NP = 1024
_B = 4
_D = 256
_K = 5
_KP = 8
_R = 128
_INF = 3.0e38


def _dgm_tile(t_ref, xp_ref, sqr_ref, sqc_ref, idx_ref, logp_ref):
    i = pl.program_id(0)
    t = jnp.exp(jnp.clip(t_ref[0, 0], -5.0, 5.0))
    ds = []
    s = jnp.zeros((_R, _NP), jnp.float32)
    for b in range(_B):
        xb_all = xp_ref[b]
        xb_rows = xp_ref[b, pl.ds(i * _R, _R), :]
        sq_rows = sqr_ref[b]
        sq_all = sqc_ref[b]
        g = jax.lax.dot_general(
            xb_rows, xb_all,
            (((1,), (1,)), ((), ())),
            preferred_element_type=jnp.float32)
        db = (sq_rows + sq_all - 2.0 * g) * t
        ds.append(db)
        s = s + db
    lane = jax.lax.broadcasted_iota(jnp.int32, (_R, _NP), 1)
    swork = jnp.where(lane < _N, s, _INF)
    kcol = jax.lax.broadcasted_iota(jnp.int32, (_R, _KP), 1)
    idx_acc = jnp.zeros((_R, _KP), jnp.int32)
    logp_acc = [jnp.zeros((_R, _KP), jnp.float32) for _ in range(_B)]
    for k in range(_K):
        idx = jnp.argmin(swork, axis=1, keepdims=True).astype(jnp.int32)
        onehot = lane == idx
        swork = jnp.where(onehot, _INF, swork)
        idx_acc = jnp.where(kcol == k, idx, idx_acc)
        for b in range(_B):
            val = jnp.sum(jnp.where(onehot, ds[b], 0.0), axis=1,
                          keepdims=True)
            logp_acc[b] = jnp.where(kcol == k, -val, logp_acc[b])
    idx_ref[...] = idx_acc
    for b in range(_B):
        logp_ref[b] = logp_acc[b]


@jax.jit
def kernel(x, A, temperature):
    xr = jnp.reshape(x[0], (_B, _N, _D))
    xp = jnp.pad(xr, ((0, 0), (0, _NP - _N), (0, 0)))
    sq = jnp.sum(xp * xp, axis=-1)
    sq_rows = sq[:, :, None]
    sq_cols = sq[:, None, :]
    t2 = jnp.reshape(temperature.astype(jnp.float32), (1, 1))
    grid = _NP // _R
    idx_pad, logp_pad = pl.pallas_call(
        _dgm_tile,
        grid=(grid,),
        in_specs=[
            pl.BlockSpec(memory_space=pltpu.SMEM),
            pl.BlockSpec((_B, _NP, _D), lambda i: (0, 0, 0)),
            pl.BlockSpec((_B, _R, 1), lambda i: (0, i, 0)),
            pl.BlockSpec((_B, 1, _NP), lambda i: (0, 0, 0)),
        ],
        out_specs=[
            pl.BlockSpec((_R, _KP), lambda i: (i, 0)),
            pl.BlockSpec((_B, _R, _KP), lambda i: (0, i, 0)),
        ],
        out_shape=[
            jax.ShapeDtypeStruct((_NP, _KP), jnp.int32),
            jax.ShapeDtypeStruct((_B, _NP, _KP), jnp.float32),
        ],
    )(t2, xp, sq_rows, sq_cols)
    indices = idx_pad[:_N, :_K]
    logprobs = logp_pad[:, :_N, :_K]
    flat_idx = indices.reshape(-1)
    src = jnp.tile(jnp.repeat(jnp.arange(_N), _K), _B)
    tfg = jnp.tile(flat_idx, _B)
    offset = jnp.repeat(jnp.arange(_B) * _N, _N * _K)
    edges = jnp.stack([src + offset, tfg + offset])
    return (x, edges, logprobs)


# speedup vs baseline: 10.9232x; 1.1774x over previous
import functools

import jax
import jax.numpy as jnp
from jax.experimental import pallas as pl
from jax.experimental.pallas import tpu as pltpu

_N = 1000
_B = 4
_D = 256
_K = 5
_KP = 8
_R = 200
_INF = 3.0e38


def _dgm_tile(t_ref, xp_ref, sqr_ref, sqc_ref, idx_ref, logp_ref):
    i = pl.program_id(0)
    t = jnp.exp(jnp.clip(t_ref[0, 0], -5.0, 5.0))
    ds = []
    s = jnp.zeros((_R, _N), jnp.float32)
    for b in range(_B):
        xb_all = xp_ref[b]
        xb_rows = xp_ref[b, pl.ds(i * _R, _R), :]
        sq_rows = sqr_ref[b]
        sq_all = sqc_ref[b]
        g = jax.lax.dot_general(
            xb_rows, xb_all,
            (((1,), (1,)), ((), ())),
            preferred_element_type=jnp.float32)
        db = (sq_rows + sq_all - 2.0 * g) * t
        ds.append(db)
        s = s + db
    lane = jax.lax.broadcasted_iota(jnp.int32, (_R, _N), 1)
    kcol = jax.lax.broadcasted_iota(jnp.int32, (_R, _KP), 1)
    swork = s
    idx_acc = jnp.zeros((_R, _KP), jnp.int32)
    logp_acc = [jnp.zeros((_R, _KP), jnp.float32) for _ in range(_B)]
    for k in range(_K):
        idx = jnp.argmin(swork, axis=1, keepdims=True).astype(jnp.int32)
        onehot = lane == idx
        swork = jnp.where(onehot, _INF, swork)
        idx_acc = jnp.where(kcol == k, idx, idx_acc)
        for b in range(_B):
            val = jnp.sum(jnp.where(onehot, ds[b], 0.0), axis=1,
                          keepdims=True)
            logp_acc[b] = jnp.where(kcol == k, -val, logp_acc[b])
    idx_ref[...] = idx_acc
    for b in range(_B):
        logp_ref[b] = logp_acc[b]


@jax.jit
def kernel(x, A, temperature):
    xr = jnp.reshape(x[0], (_B, _N, _D))
    sq = jnp.sum(xr * xr, axis=-1)
    sq_rows = sq[:, :, None]
    sq_cols = sq[:, None, :]
    t2 = jnp.reshape(temperature.astype(jnp.float32), (1, 1))
    grid = _N // _R
    indices, logp = pl.pallas_call(
        _dgm_tile,
        grid=(grid,),
        in_specs=[
            pl.BlockSpec(memory_space=pltpu.SMEM),
            pl.BlockSpec((_B, _N, _D), lambda i: (0, 0, 0)),
            pl.BlockSpec((_B, _R, 1), lambda i: (0, i, 0)),
            pl.BlockSpec((_B, 1, _N), lambda i: (0, 0, 0)),
        ],
        out_specs=[
            pl.BlockSpec((_R, _KP), lambda i: (i, 0)),
            pl.BlockSpec((_B, _R, _KP), lambda i: (0, i, 0)),
        ],
        out_shape=[
            jax.ShapeDtypeStruct((_N, _KP), jnp.int32),
            jax.ShapeDtypeStruct((_B, _N, _KP), jnp.float32),
        ],
    )(t2, xr, sq_rows, sq_cols)
    indices = indices[:, :_K]
    logprobs = logp[:, :, :_K]
    flat_idx = indices.reshape(-1)
    src = jnp.tile(jnp.repeat(jnp.arange(_N), _K), _B)
    tfg = jnp.tile(flat_idx, _B)
    offset = jnp.repeat(jnp.arange(_B) * _N, _N * _K)
    edges = jnp.stack([src + offset, tfg + offset])
    return (x, edges, logprobs)
